# baseline (speedup 1.0000x reference)
import functools

import jax
import jax.numpy as jnp
from jax import lax
from jax.experimental import pallas as pl
from jax.experimental.pallas import tpu as pltpu
from jax.experimental.pallas import tpu_sc as plsc

_B, _S, _D = 16, 2048, 512
_SLOTS = 4
_ROWS = _B * _SLOTS
_EXT = _ROWS + 2 * _SLOTS


@functools.partial(
    pl.kernel,
    out_type=jax.ShapeDtypeStruct((_EXT, _D), jnp.float32),
    mesh=plsc.VectorSubcoreMesh(core_axis_name="c", subcore_axis_name="s"),
    scratch_types=[
        pltpu.VMEM((16,), jnp.int32),
        pltpu.VMEM((16,), jnp.int32),
        pltpu.VMEM((16,), jnp.int32),
        pltpu.VMEM((16, _D), jnp.float32),
        pltpu.VMEM((16, _D), jnp.float32),
        pltpu.SemaphoreType.DMA,
        pltpu.SemaphoreType.DMA,
        pltpu.SemaphoreType.DMA,
    ],
)
def _encode_sc(ctx_hbm, sb_hbm, sl_hbm, bl_hbm, pad_hbm, out_hbm,
               sl_v, bl_v, tok_v, rows_v, pad_v, sem0, sem_a, sem_b):
    wid = lax.axis_index("s") * 2 + lax.axis_index("c")

    @pl.when(wid < _SLOTS)
    def _():
        j = wid
        pad_cp = pltpu.async_copy(pad_hbm, pad_v, sem_b)
        pltpu.sync_copy(sl_hbm, sl_v)
        pltpu.sync_copy(bl_hbm, bl_v)
        lane = lax.iota(jnp.int32, 16)
        is_buf = j == _SLOTS - 1
        length = jnp.where(is_buf, bl_v[...], sl_v[...])
        pos = length + jnp.where(is_buf, -1, j - 3)
        valid = pos >= 0
        posc = jnp.maximum(pos, 0)
        sb_idx = lane * _S + jnp.where(is_buf, _B * _S, 0) + posc
        pltpu.async_copy(sb_hbm.at[sb_idx], tok_v, sem0).wait()
        row_idx = lane * _S + tok_v[...]
        pltpu.async_copy(ctx_hbm.at[row_idx], rows_v, sem0).wait()
        dst_a = jnp.where(valid, lane * _SLOTS + j, _ROWS + 2 * j)
        cp_a = pltpu.async_copy(rows_v, out_hbm.at[dst_a], sem_a)
        pad_cp.wait()
        dst_b = jnp.where(valid, _ROWS + 2 * j + 1, lane * _SLOTS + j)
        cp_b = pltpu.async_copy(pad_v, out_hbm.at[dst_b], sem_b)
        cp_a.wait()
        cp_b.wait()


def kernel(contextualized_input_batch, stacks, buffers, stack_lengths,
           buffer_lengths, padding):
    ctx = contextualized_input_batch.reshape(_B * _S, _D)
    sb = jnp.concatenate(
        [stacks.astype(jnp.int32), buffers.astype(jnp.int32)], axis=0
    ).reshape(2 * _B * _S)
    sl = stack_lengths.astype(jnp.int32)
    bl = buffer_lengths.astype(jnp.int32)
    pad16 = jnp.broadcast_to(padding, (16, _D))
    out_ext = _encode_sc(ctx, sb, sl, bl, pad16)
    return out_ext[:_ROWS].reshape(_B, _SLOTS * _D)


# speedup vs baseline: 1.0161x; 1.0161x over previous
import functools

import jax
import jax.numpy as jnp
from jax import lax
from jax.experimental import pallas as pl
from jax.experimental.pallas import tpu as pltpu
from jax.experimental.pallas import tpu_sc as plsc

_B, _S, _D = 16, 2048, 512
_SLOTS = 4
_ROWS = _B * _SLOTS


@functools.partial(
    pl.kernel,
    out_type=jax.ShapeDtypeStruct((_ROWS, _D), jnp.float32),
    mesh=plsc.VectorSubcoreMesh(core_axis_name="c", subcore_axis_name="s"),
    scratch_types=[
        pltpu.VMEM((16,), jnp.int32),
        pltpu.VMEM((16,), jnp.int32),
        pltpu.VMEM((16,), jnp.int32),
        pltpu.VMEM((16, _D), jnp.float32),
        pltpu.SemaphoreType.DMA,
        pltpu.SemaphoreType.DMA,
    ],
)
def _encode_sc(ctx_hbm, st_hbm, bu_hbm, sl_hbm, bl_hbm, pad_hbm, out_hbm,
               sl_v, bl_v, tok_v, rows_v, sem0, sem1):
    wid = lax.axis_index("s") * 2 + lax.axis_index("c")

    @pl.when(wid < _SLOTS)
    def _():
        j = wid
        cp_sl = pltpu.async_copy(sl_hbm, sl_v, sem0)
        cp_bl = pltpu.async_copy(bl_hbm, bl_v, sem1)
        cp_sl.wait()
        cp_bl.wait()
        lane = lax.iota(jnp.int32, 16)
        is_buf = j == _SLOTS - 1
        length = jnp.where(is_buf, bl_v[...], sl_v[...])
        pos = length + jnp.where(is_buf, -1, j - 3)
        idx = lane * _S + jnp.maximum(pos, 0)

        @pl.when(jnp.logical_not(is_buf))
        def _():
            pltpu.async_copy(st_hbm.at[idx], tok_v, sem0).wait()

        @pl.when(is_buf)
        def _():
            pltpu.async_copy(bu_hbm.at[idx], tok_v, sem0).wait()

        row_idx = lane * _S + tok_v[...]
        pltpu.async_copy(ctx_hbm.at[row_idx], rows_v, sem0).wait()
        for b in range(16):
            @pl.when(pos[b] < 0)
            def _():
                pltpu.sync_copy(pad_hbm, rows_v.at[b])
        pltpu.async_copy(rows_v, out_hbm.at[lane * _SLOTS + j], sem0).wait()


def kernel(contextualized_input_batch, stacks, buffers, stack_lengths,
           buffer_lengths, padding):
    ctx = contextualized_input_batch.reshape(_B * _S, _D)
    st = stacks.astype(jnp.int32).reshape(_B * _S)
    bu = buffers.astype(jnp.int32).reshape(_B * _S)
    sl = stack_lengths.astype(jnp.int32)
    bl = buffer_lengths.astype(jnp.int32)
    out = _encode_sc(ctx, st, bu, sl, bl, padding)
    return out.reshape(_B, _SLOTS * _D)


# speedup vs baseline: 1.1086x; 1.0910x over previous
import functools

import jax
import jax.numpy as jnp
from jax import lax
from jax.experimental import pallas as pl
from jax.experimental.pallas import tpu as pltpu
from jax.experimental.pallas import tpu_sc as plsc

_B, _S, _D = 16, 2048, 512
_SLOTS = 4
_ROWS = _B * _SLOTS


@functools.partial(
    pl.kernel,
    out_type=jax.ShapeDtypeStruct((_ROWS, _D), jnp.float32),
    mesh=plsc.VectorSubcoreMesh(core_axis_name="c", subcore_axis_name="s",
                                num_cores=1),
    scratch_types=[
        pltpu.VMEM((16,), jnp.int32),
        pltpu.VMEM((16,), jnp.int32),
        pltpu.VMEM((16,), jnp.int32),
        pltpu.VMEM((16, _D), jnp.float32),
        pltpu.SemaphoreType.DMA,
        pltpu.SemaphoreType.DMA,
    ],
)
def _encode_sc(ctx_hbm, st_hbm, bu_hbm, sl_hbm, bl_hbm, pad_hbm, out_hbm,
               sl_v, bl_v, tok_v, rows_v, sem0, sem1):
    wid = lax.axis_index("s")

    @pl.when(wid < _SLOTS)
    def _():
        j = wid
        cp_sl = pltpu.async_copy(sl_hbm, sl_v, sem0)
        cp_bl = pltpu.async_copy(bl_hbm, bl_v, sem1)
        cp_sl.wait()
        cp_bl.wait()
        lane = lax.iota(jnp.int32, 16)
        is_buf = j == _SLOTS - 1
        length = jnp.where(is_buf, bl_v[...], sl_v[...])
        pos = length + jnp.where(is_buf, -1, j - 3)
        idx = lane * _S + jnp.maximum(pos, 0)

        @pl.when(jnp.logical_not(is_buf))
        def _():
            pltpu.async_copy(st_hbm.at[idx], tok_v, sem0).wait()

        @pl.when(is_buf)
        def _():
            pltpu.async_copy(bu_hbm.at[idx], tok_v, sem0).wait()

        row_idx = lane * _S + tok_v[...]
        pltpu.async_copy(ctx_hbm.at[row_idx], rows_v, sem0).wait()
        for b in range(16):
            @pl.when(pos[b] < 0)
            def _():
                pltpu.sync_copy(pad_hbm, rows_v.at[b])
        pltpu.async_copy(rows_v, out_hbm.at[lane * _SLOTS + j], sem0).wait()


def kernel(contextualized_input_batch, stacks, buffers, stack_lengths,
           buffer_lengths, padding):
    ctx = contextualized_input_batch.reshape(_B * _S, _D)
    st = stacks.astype(jnp.int32).reshape(_B * _S)
    bu = buffers.astype(jnp.int32).reshape(_B * _S)
    sl = stack_lengths.astype(jnp.int32)
    bl = buffer_lengths.astype(jnp.int32)
    out = _encode_sc(ctx, st, bu, sl, bl, padding)
    return out.reshape(_B, _SLOTS * _D)
